# Initial kernel scaffold; baseline (speedup 1.0000x reference)
import jax
import jax.numpy as jnp
from jax.experimental import pallas as pl


def kernel(x, edge_index, W1, b1, W2, b2):
    raise NotImplementedError("write your pallas kernel here")


# speedup vs baseline: 32.6992x; 32.6992x over previous
import functools

import jax
import jax.numpy as jnp
from jax import lax
from jax.experimental import pallas as pl
from jax.experimental.pallas import tpu as pltpu
from jax.experimental.pallas import tpu_sc as plsc

N = 10000
E = 320000
FIN = 128
D = 16
C = 11

NC = 2
NS = 16
NW = NC * NS
L = 16

EPT_RAW = E // NW
B = 128
NB = -(-EPT_RAW // B)
EPT = NB * B
NPAD = 10112
RPT = NPAD // NS
SINK = N

_MESH = plsc.VectorSubcoreMesh(core_axis_name="c", subcore_axis_name="s")


def _sc_deg_body(dst_hbm, hist_out, dst_v, hist_v):
    c = lax.axis_index("c")
    s = lax.axis_index("s")
    wid = c * NS + s
    pltpu.sync_copy(dst_hbm.at[wid], dst_v)
    zero = jnp.zeros((L,), jnp.float32)
    ones = jnp.ones((L,), jnp.float32)

    def zbody(i, carry):
        hist_v[pl.ds(i * L, L)] = zero
        return carry

    lax.fori_loop(0, NPAD // L, zbody, 0)

    def ebody(i, carry):
        idx = dst_v[pl.ds(i * L, L)]
        plsc.addupdate_scatter(hist_v, [idx], ones)
        return carry

    lax.fori_loop(0, EPT // L, ebody, 0)
    pltpu.sync_copy(hist_v, hist_out.at[wid])


_sc_deg = functools.partial(
    pl.kernel,
    out_type=jax.ShapeDtypeStruct((NW, NPAD), jnp.float32),
    mesh=_MESH,
    compiler_params=pltpu.CompilerParams(needs_layout_passes=False),
    scratch_types=[
        pltpu.VMEM((EPT,), jnp.int32),
        pltpu.VMEM((NPAD,), jnp.float32),
    ],
)(_sc_deg_body)


def _sc_scatter_body(g_hbm, src_hbm, dst_hbm, out_hbm,
                     src_v, dst_v, row_v, zrow_v, acc_sh, sem):
    c = lax.axis_index("c")
    s = lax.axis_index("s")
    wid = c * NS + s
    pltpu.sync_copy(src_hbm.at[wid], src_v)
    pltpu.sync_copy(dst_hbm.at[wid], dst_v)
    zero = jnp.zeros((L,), jnp.float32)

    def zbody(i, carry):
        zrow_v[i, :] = zero
        return carry

    lax.fori_loop(0, RPT, zbody, 0)
    pltpu.sync_copy(zrow_v, acc_sh.at[pl.ds(s * RPT, RPT)])
    plsc.subcore_barrier()

    def ebody(j, carry):
        pltpu.async_copy(g_hbm.at[src_v.at[j]], row_v, sem).wait()
        pltpu.sync_copy(row_v, acc_sh.at[dst_v.at[j]], add=True)
        return carry

    lax.fori_loop(0, NB, ebody, 0)
    plsc.subcore_barrier()
    pltpu.sync_copy(acc_sh.at[pl.ds(s * RPT, RPT)],
                    out_hbm.at[c, pl.ds(s * RPT, RPT)])


_sc_scatter = functools.partial(
    pl.kernel,
    out_type=jax.ShapeDtypeStruct((NC, NPAD, D), jnp.float32),
    mesh=_MESH,
    compiler_params=pltpu.CompilerParams(needs_layout_passes=False,
                                         use_tc_tiling_on_sc=False),
    scratch_types=[
        pltpu.VMEM((NB, B), jnp.int32),
        pltpu.VMEM((NB, B), jnp.int32),
        pltpu.VMEM((B, D), jnp.float32),
        pltpu.VMEM((RPT, D), jnp.float32),
        pltpu.VMEM_SHARED((NPAD, D), jnp.float32),
        pltpu.SemaphoreType.DMA,
    ],
)(_sc_scatter_body)


def _tc_prep1_body(hT_ref, x_ref, W1_ref, dinv_ref, g1_ref):
    deg = jnp.sum(hT_ref[...], axis=1, keepdims=True) + 1.0
    dinv = lax.rsqrt(deg)
    dinv_ref[...] = dinv
    h = jnp.dot(x_ref[...], W1_ref[...], preferred_element_type=jnp.float32)
    g1_ref[...] = dinv * h


def _tc_mid_body(s1_ref, g1_ref, dinv_ref, b1_ref, W2_ref, g2_ref):
    dinv = dinv_ref[...]
    ssum = s1_ref[0] + s1_ref[1] + g1_ref[...]
    out1 = jnp.maximum(dinv * ssum + b1_ref[...], 0.0)
    g2_ref[...] = dinv * jnp.dot(out1, W2_ref[...],
                                 preferred_element_type=jnp.float32)


def _tc_final_body(s2_ref, g2_ref, dinv_ref, b2_ref, o_ref):
    o = dinv_ref[...] * (s2_ref[0] + s2_ref[1] + g2_ref[...]) + b2_ref[...]
    mask = lax.broadcasted_iota(jnp.int32, (NPAD, D), 1) < C
    om = jnp.where(mask, o, -jnp.inf)
    m = jnp.max(om, axis=1, keepdims=True)
    z = om - m
    e = jnp.where(mask, jnp.exp(z), 0.0)
    o_ref[...] = z - jnp.log(jnp.sum(e, axis=1, keepdims=True))


_tc_prep1 = pl.pallas_call(
    _tc_prep1_body,
    out_shape=(jax.ShapeDtypeStruct((NPAD, 1), jnp.float32),
               jax.ShapeDtypeStruct((NPAD, D), jnp.float32)),
)

_tc_mid = pl.pallas_call(
    _tc_mid_body,
    out_shape=jax.ShapeDtypeStruct((NPAD, D), jnp.float32),
)

_tc_final = pl.pallas_call(
    _tc_final_body,
    out_shape=jax.ShapeDtypeStruct((NPAD, D), jnp.float32),
)


def kernel(x, edge_index, W1, b1, W2, b2):
    src = edge_index[0].reshape(NW, EPT_RAW)
    dst = edge_index[1].reshape(NW, EPT_RAW)
    pad = EPT - EPT_RAW
    src_p = jnp.pad(src, ((0, 0), (0, pad)))
    dst_p = jnp.pad(dst, ((0, 0), (0, pad)), constant_values=SINK)
    src_t = src_p.reshape(NW, NB, B)
    dst_t = dst_p.reshape(NW, NB, B)

    xp = jnp.pad(x, ((0, NPAD - N), (0, 0)))
    W2p = jnp.pad(W2, ((0, 0), (0, D - C)))
    b1r = b1.reshape(1, D)
    b2r = jnp.pad(b2, (0, D - C)).reshape(1, D)

    hists = _sc_deg(dst_p)
    dinv, g1 = _tc_prep1(hists.T, xp, W1)
    s1 = _sc_scatter(g1, src_t, dst_t)
    g2 = _tc_mid(s1, g1, dinv, b1r, W2p)
    s2 = _sc_scatter(g2, src_t, dst_t)
    o = _tc_final(s2, g2, dinv, b2r)
    return o[:N, :C]


# speedup vs baseline: 40.2685x; 1.2315x over previous
import functools

import jax
import jax.numpy as jnp
from jax import lax
from jax.experimental import pallas as pl
from jax.experimental.pallas import tpu as pltpu
from jax.experimental.pallas import tpu_sc as plsc

N = 10000
E = 320000
FIN = 128
D = 16
C = 11

NC = 2
NS = 16
NW = NC * NS
L = 16

EPT_RAW = E // NW
B = 128
K = 8
NB = 80
NCH = NB // K
EPT = NB * B
NPAD = 10112
RPT = NPAD // NS
SINK = N

_MESH = plsc.VectorSubcoreMesh(core_axis_name="c", subcore_axis_name="s")


def _sc_deg_body(dst_hbm, hist_out, dst_v, hist_v):
    c = lax.axis_index("c")
    s = lax.axis_index("s")
    wid = c * NS + s
    pltpu.sync_copy(dst_hbm.at[wid], dst_v)
    zero = jnp.zeros((L,), jnp.float32)
    ones = jnp.ones((L,), jnp.float32)

    def zbody(i, carry):
        hist_v[pl.ds(i * L, L)] = zero
        return carry

    lax.fori_loop(0, NPAD // L, zbody, 0)

    def ebody(i, carry):
        idx = dst_v[pl.ds(i * L, L)]
        plsc.addupdate_scatter(hist_v, [idx], ones)
        return carry

    lax.fori_loop(0, EPT // L, ebody, 0)
    pltpu.sync_copy(hist_v, hist_out.at[wid])


_sc_deg = functools.partial(
    pl.kernel,
    out_type=jax.ShapeDtypeStruct((NW, NPAD), jnp.float32),
    mesh=_MESH,
    compiler_params=pltpu.CompilerParams(needs_layout_passes=False),
    scratch_types=[
        pltpu.VMEM((EPT,), jnp.int32),
        pltpu.VMEM((NPAD,), jnp.float32),
    ],
)(_sc_deg_body)


def _sc_scatter_body(g_hbm, src_hbm, dst_hbm, out_hbm,
                     src_v, dst_v, ring_v, zrow_v, acc_sh, gsem, ssem):
    c = lax.axis_index("c")
    s = lax.axis_index("s")
    wid = c * NS + s
    pltpu.sync_copy(src_hbm.at[wid], src_v)
    pltpu.sync_copy(dst_hbm.at[wid], dst_v)
    zero = jnp.zeros((L,), jnp.float32)

    def zbody(i, carry):
        zrow_v[i, :] = zero
        return carry

    lax.fori_loop(0, RPT, zbody, 0)
    pltpu.sync_copy(zrow_v, acc_sh.at[pl.ds(s * RPT, RPT)])
    plsc.subcore_barrier()

    def _gathers(ch, grp):
        for b in range(K):
            pltpu.async_copy(g_hbm.at[src_v.at[ch * K + b]],
                             ring_v.at[grp, b], gsem)

    def _gather_wait():
        pltpu.make_async_copy(g_hbm.at[src_v.at[0]], ring_v.at[0, 0],
                              gsem).wait()

    def _scatter_wait():
        pltpu.make_async_copy(ring_v.at[0, 0], acc_sh.at[dst_v.at[0]],
                              ssem).wait()

    _gathers(0, 0)

    def cbody(ch, carry):
        grp = lax.rem(ch, 2)
        for _ in range(K):
            _gather_wait()
        for b in range(K):
            pltpu.async_copy(ring_v.at[grp, b], acc_sh.at[dst_v.at[ch * K + b]],
                             ssem, add=True)

        @pl.when(ch + 1 < NCH)
        def _():
            _gathers(ch + 1, 1 - grp)

        for _ in range(K):
            _scatter_wait()
        return carry

    lax.fori_loop(0, NCH, cbody, 0)
    plsc.subcore_barrier()
    pltpu.sync_copy(acc_sh.at[pl.ds(s * RPT, RPT)],
                    out_hbm.at[c, pl.ds(s * RPT, RPT)])


_sc_scatter = functools.partial(
    pl.kernel,
    out_type=jax.ShapeDtypeStruct((NC, NPAD, D), jnp.float32),
    mesh=_MESH,
    compiler_params=pltpu.CompilerParams(needs_layout_passes=False,
                                         use_tc_tiling_on_sc=False),
    scratch_types=[
        pltpu.VMEM((NB, B), jnp.int32),
        pltpu.VMEM((NB, B), jnp.int32),
        pltpu.VMEM((2, K, B, D), jnp.float32),
        pltpu.VMEM((RPT, D), jnp.float32),
        pltpu.VMEM_SHARED((NPAD, D), jnp.float32),
        pltpu.SemaphoreType.DMA,
        pltpu.SemaphoreType.DMA,
    ],
)(_sc_scatter_body)


def _tc_prep1_body(hT_ref, x_ref, W1_ref, dinv_ref, g1_ref):
    deg = jnp.sum(hT_ref[...], axis=1, keepdims=True) + 1.0
    dinv = lax.rsqrt(deg)
    dinv_ref[...] = dinv
    h = jnp.dot(x_ref[...], W1_ref[...], preferred_element_type=jnp.float32)
    g1_ref[...] = dinv * h


def _tc_mid_body(s1_ref, g1_ref, dinv_ref, b1_ref, W2_ref, g2_ref):
    dinv = dinv_ref[...]
    ssum = s1_ref[0] + s1_ref[1] + g1_ref[...]
    out1 = jnp.maximum(dinv * ssum + b1_ref[...], 0.0)
    g2_ref[...] = dinv * jnp.dot(out1, W2_ref[...],
                                 preferred_element_type=jnp.float32)


def _tc_final_body(s2_ref, g2_ref, dinv_ref, b2_ref, o_ref):
    o = dinv_ref[...] * (s2_ref[0] + s2_ref[1] + g2_ref[...]) + b2_ref[...]
    mask = lax.broadcasted_iota(jnp.int32, (NPAD, D), 1) < C
    om = jnp.where(mask, o, -jnp.inf)
    m = jnp.max(om, axis=1, keepdims=True)
    z = om - m
    e = jnp.where(mask, jnp.exp(z), 0.0)
    o_ref[...] = z - jnp.log(jnp.sum(e, axis=1, keepdims=True))


_tc_prep1 = pl.pallas_call(
    _tc_prep1_body,
    out_shape=(jax.ShapeDtypeStruct((NPAD, 1), jnp.float32),
               jax.ShapeDtypeStruct((NPAD, D), jnp.float32)),
)

_tc_mid = pl.pallas_call(
    _tc_mid_body,
    out_shape=jax.ShapeDtypeStruct((NPAD, D), jnp.float32),
)

_tc_final = pl.pallas_call(
    _tc_final_body,
    out_shape=jax.ShapeDtypeStruct((NPAD, D), jnp.float32),
)


def kernel(x, edge_index, W1, b1, W2, b2):
    src = edge_index[0].reshape(NW, EPT_RAW)
    dst = edge_index[1].reshape(NW, EPT_RAW)
    pad = EPT - EPT_RAW
    src_p = jnp.pad(src, ((0, 0), (0, pad)))
    dst_p = jnp.pad(dst, ((0, 0), (0, pad)), constant_values=SINK)
    src_t = src_p.reshape(NW, NB, B)
    dst_t = dst_p.reshape(NW, NB, B)

    xp = jnp.pad(x, ((0, NPAD - N), (0, 0)))
    W2p = jnp.pad(W2, ((0, 0), (0, D - C)))
    b1r = b1.reshape(1, D)
    b2r = jnp.pad(b2, (0, D - C)).reshape(1, D)

    hists = _sc_deg(dst_p)
    dinv, g1 = _tc_prep1(hists.T, xp, W1)
    s1 = _sc_scatter(g1, src_t, dst_t)
    g2 = _tc_mid(s1, g1, dinv, b1r, W2p)
    s2 = _sc_scatter(g2, src_t, dst_t)
    o = _tc_final(s2, g2, dinv, b2r)
    return o[:N, :C]


# speedup vs baseline: 58.1023x; 1.4429x over previous
import functools

import jax
import jax.numpy as jnp
from jax import lax
from jax.experimental import pallas as pl
from jax.experimental.pallas import tpu as pltpu
from jax.experimental.pallas import tpu_sc as plsc

N = 10000
E = 320000
FIN = 128
D = 16
C = 11

NC = 2
NS = 16
NW = NC * NS
L = 16

EPT_RAW = E // NW
B = 128
K = 8
NB = 80
NCH = NB // K
EPT = NB * B
NPAD = 10112
RPT = NPAD // NS
SINK = N

_MESH = plsc.VectorSubcoreMesh(core_axis_name="c", subcore_axis_name="s")


def _sc_deg_body(dst_hbm, hist_out, dst_v, hist_v):
    c = lax.axis_index("c")
    s = lax.axis_index("s")
    wid = c * NS + s
    pltpu.sync_copy(dst_hbm.at[wid], dst_v)
    zero = jnp.zeros((L,), jnp.float32)
    ones = jnp.ones((L,), jnp.float32)

    def zbody(i, carry):
        hist_v[pl.ds(i * L, L)] = zero
        return carry

    lax.fori_loop(0, NPAD // L, zbody, 0)

    def ebody(i, carry):
        idx = dst_v[pl.ds(i * L, L)]
        plsc.addupdate_scatter(hist_v, [idx], ones)
        return carry

    lax.fori_loop(0, EPT // L, ebody, 0)
    pltpu.sync_copy(hist_v, hist_out.at[wid])


_sc_deg = functools.partial(
    pl.kernel,
    out_type=jax.ShapeDtypeStruct((NW, NPAD), jnp.float32),
    mesh=_MESH,
    compiler_params=pltpu.CompilerParams(needs_layout_passes=False),
    scratch_types=[
        pltpu.VMEM((EPT,), jnp.int32),
        pltpu.VMEM((NPAD,), jnp.float32),
    ],
)(_sc_deg_body)


def _sc_scatter_body(g_hbm, src_hbm, dst_hbm, out_hbm,
                     src_v, dst_v, ring_v, zrow_v, acc_sh, gt_sh, gsem, ssem):
    c = lax.axis_index("c")
    s = lax.axis_index("s")
    wid = c * NS + s
    pltpu.sync_copy(src_hbm.at[wid], src_v)
    pltpu.sync_copy(dst_hbm.at[wid], dst_v)
    pltpu.sync_copy(g_hbm.at[pl.ds(s * RPT, RPT)],
                    gt_sh.at[pl.ds(s * RPT, RPT)])
    zero = jnp.zeros((L,), jnp.float32)

    def zbody(i, carry):
        zrow_v[i, :] = zero
        return carry

    lax.fori_loop(0, RPT, zbody, 0)
    pltpu.sync_copy(zrow_v, acc_sh.at[pl.ds(s * RPT, RPT)])
    plsc.subcore_barrier()

    def _gathers(ch, grp):
        for b in range(K):
            pltpu.async_copy(gt_sh.at[src_v.at[ch * K + b]],
                             ring_v.at[grp, b], gsem)

    def _gather_wait():
        pltpu.make_async_copy(gt_sh.at[src_v.at[0]], ring_v.at[0, 0],
                              gsem).wait()

    def _scatter_wait():
        pltpu.make_async_copy(ring_v.at[0, 0], acc_sh.at[dst_v.at[0]],
                              ssem).wait()

    _gathers(0, 0)

    def cbody(ch, carry):
        grp = lax.rem(ch, 2)
        for _ in range(K):
            _gather_wait()
        for b in range(K):
            pltpu.async_copy(ring_v.at[grp, b], acc_sh.at[dst_v.at[ch * K + b]],
                             ssem, add=True)

        @pl.when(ch + 1 < NCH)
        def _():
            _gathers(ch + 1, 1 - grp)

        for _ in range(K):
            _scatter_wait()
        return carry

    lax.fori_loop(0, NCH, cbody, 0)
    plsc.subcore_barrier()
    pltpu.sync_copy(acc_sh.at[pl.ds(s * RPT, RPT)],
                    out_hbm.at[c, pl.ds(s * RPT, RPT)])


_sc_scatter = functools.partial(
    pl.kernel,
    out_type=jax.ShapeDtypeStruct((NC, NPAD, D), jnp.float32),
    mesh=_MESH,
    compiler_params=pltpu.CompilerParams(needs_layout_passes=False,
                                         use_tc_tiling_on_sc=False),
    scratch_types=[
        pltpu.VMEM((NB, B), jnp.int32),
        pltpu.VMEM((NB, B), jnp.int32),
        pltpu.VMEM((2, K, B, D), jnp.float32),
        pltpu.VMEM((RPT, D), jnp.float32),
        pltpu.VMEM_SHARED((NPAD, D), jnp.float32),
        pltpu.VMEM_SHARED((NPAD, D), jnp.float32),
        pltpu.SemaphoreType.DMA,
        pltpu.SemaphoreType.DMA,
    ],
)(_sc_scatter_body)


def _tc_prep1_body(hT_ref, x_ref, W1_ref, dinv_ref, g1_ref):
    deg = jnp.sum(hT_ref[...], axis=1, keepdims=True) + 1.0
    dinv = lax.rsqrt(deg)
    dinv_ref[...] = dinv
    h = jnp.dot(x_ref[...], W1_ref[...], preferred_element_type=jnp.float32)
    g1_ref[...] = dinv * h


def _tc_mid_body(s1_ref, g1_ref, dinv_ref, b1_ref, W2_ref, g2_ref):
    dinv = dinv_ref[...]
    ssum = s1_ref[0] + s1_ref[1] + g1_ref[...]
    out1 = jnp.maximum(dinv * ssum + b1_ref[...], 0.0)
    g2_ref[...] = dinv * jnp.dot(out1, W2_ref[...],
                                 preferred_element_type=jnp.float32)


def _tc_final_body(s2_ref, g2_ref, dinv_ref, b2_ref, o_ref):
    o = dinv_ref[...] * (s2_ref[0] + s2_ref[1] + g2_ref[...]) + b2_ref[...]
    mask = lax.broadcasted_iota(jnp.int32, (NPAD, D), 1) < C
    om = jnp.where(mask, o, -jnp.inf)
    m = jnp.max(om, axis=1, keepdims=True)
    z = om - m
    e = jnp.where(mask, jnp.exp(z), 0.0)
    o_ref[...] = z - jnp.log(jnp.sum(e, axis=1, keepdims=True))


_tc_prep1 = pl.pallas_call(
    _tc_prep1_body,
    out_shape=(jax.ShapeDtypeStruct((NPAD, 1), jnp.float32),
               jax.ShapeDtypeStruct((NPAD, D), jnp.float32)),
)

_tc_mid = pl.pallas_call(
    _tc_mid_body,
    out_shape=jax.ShapeDtypeStruct((NPAD, D), jnp.float32),
)

_tc_final = pl.pallas_call(
    _tc_final_body,
    out_shape=jax.ShapeDtypeStruct((NPAD, D), jnp.float32),
)


def kernel(x, edge_index, W1, b1, W2, b2):
    src = edge_index[0].reshape(NW, EPT_RAW)
    dst = edge_index[1].reshape(NW, EPT_RAW)
    pad = EPT - EPT_RAW
    src_p = jnp.pad(src, ((0, 0), (0, pad)))
    dst_p = jnp.pad(dst, ((0, 0), (0, pad)), constant_values=SINK)
    src_t = src_p.reshape(NW, NB, B)
    dst_t = dst_p.reshape(NW, NB, B)

    xp = jnp.pad(x, ((0, NPAD - N), (0, 0)))
    W2p = jnp.pad(W2, ((0, 0), (0, D - C)))
    b1r = b1.reshape(1, D)
    b2r = jnp.pad(b2, (0, D - C)).reshape(1, D)

    hists = _sc_deg(dst_p)
    dinv, g1 = _tc_prep1(hists.T, xp, W1)
    s1 = _sc_scatter(g1, src_t, dst_t)
    g2 = _tc_mid(s1, g1, dinv, b1r, W2p)
    s2 = _sc_scatter(g2, src_t, dst_t)
    o = _tc_final(s2, g2, dinv, b2r)
    return o[:N, :C]


# speedup vs baseline: 58.2799x; 1.0031x over previous
import functools

import jax
import jax.numpy as jnp
from jax import lax
from jax.experimental import pallas as pl
from jax.experimental.pallas import tpu as pltpu
from jax.experimental.pallas import tpu_sc as plsc

N = 10000
E = 320000
FIN = 128
D = 16
C = 11

NC = 2
NS = 16
NW = NC * NS
L = 16

EPT_RAW = E // NW
B = 128
K = 4
G = 4
NB = 80
NCH = NB // K
EPT = NB * B
NPAD = 10112
RPT = NPAD // NS
SINK = N

_MESH = plsc.VectorSubcoreMesh(core_axis_name="c", subcore_axis_name="s")


def _sc_deg_body(dst_hbm, hist_out, dst_v, hist_v):
    c = lax.axis_index("c")
    s = lax.axis_index("s")
    wid = c * NS + s
    pltpu.sync_copy(dst_hbm.at[wid], dst_v)
    zero = jnp.zeros((L,), jnp.float32)
    ones = jnp.ones((L,), jnp.float32)

    def zbody(i, carry):
        hist_v[pl.ds(i * L, L)] = zero
        return carry

    lax.fori_loop(0, NPAD // L, zbody, 0)

    def ebody(i, carry):
        idx = dst_v[pl.ds(i * L, L)]
        plsc.addupdate_scatter(hist_v, [idx], ones)
        return carry

    lax.fori_loop(0, EPT // L, ebody, 0)
    pltpu.sync_copy(hist_v, hist_out.at[wid])


_sc_deg = functools.partial(
    pl.kernel,
    out_type=jax.ShapeDtypeStruct((NW, NPAD), jnp.float32),
    mesh=_MESH,
    compiler_params=pltpu.CompilerParams(needs_layout_passes=False),
    scratch_types=[
        pltpu.VMEM((EPT,), jnp.int32),
        pltpu.VMEM((NPAD,), jnp.float32),
    ],
)(_sc_deg_body)


def _sc_scatter_body(g_hbm, src_hbm, dst_hbm, out_hbm,
                     src_v, dst_v, ring_v, zrow_v, acc_sh, gt_sh,
                     gs0, gs1, gs2, gs3, ss0, ss1, ss2, ss3):
    c = lax.axis_index("c")
    s = lax.axis_index("s")
    wid = c * NS + s
    gsems = [gs0, gs1, gs2, gs3]
    ssems = [ss0, ss1, ss2, ss3]
    pltpu.sync_copy(src_hbm.at[wid], src_v)
    pltpu.sync_copy(dst_hbm.at[wid], dst_v)
    pltpu.sync_copy(g_hbm.at[pl.ds(s * RPT, RPT)],
                    gt_sh.at[pl.ds(s * RPT, RPT)])
    zero = jnp.zeros((L,), jnp.float32)

    def zbody(i, carry):
        zrow_v[i, :] = zero
        return carry

    lax.fori_loop(0, RPT, zbody, 0)
    pltpu.sync_copy(zrow_v, acc_sh.at[pl.ds(s * RPT, RPT)])
    plsc.subcore_barrier()

    def _gathers(ch):
        grp = ch % G
        for b in range(K):
            pltpu.async_copy(gt_sh.at[src_v.at[ch * K + b]],
                             ring_v.at[grp, b], gsems[grp])

    def _gather_wait(ch):
        for _ in range(K):
            pltpu.make_async_copy(gt_sh.at[src_v.at[0]], ring_v.at[0, 0],
                                  gsems[ch % G]).wait()

    def _scatter_wait(ch):
        for _ in range(K):
            pltpu.make_async_copy(ring_v.at[0, 0], acc_sh.at[dst_v.at[0]],
                                  ssems[ch % G]).wait()

    for ch in range(min(G - 1, NCH)):
        _gathers(ch)
    for ch in range(NCH):
        _gather_wait(ch)
        grp = ch % G
        for b in range(K):
            pltpu.async_copy(ring_v.at[grp, b], acc_sh.at[dst_v.at[ch * K + b]],
                             ssems[grp], add=True)
        nxt = ch + G - 1
        if nxt < NCH:
            if ch >= 1:
                _scatter_wait(ch - 1)
            _gathers(nxt)
    for ch in range(max(0, NCH - G), NCH):
        _scatter_wait(ch)
    plsc.subcore_barrier()
    pltpu.sync_copy(acc_sh.at[pl.ds(s * RPT, RPT)],
                    out_hbm.at[c, pl.ds(s * RPT, RPT)])


_sc_scatter = functools.partial(
    pl.kernel,
    out_type=jax.ShapeDtypeStruct((NC, NPAD, D), jnp.float32),
    mesh=_MESH,
    compiler_params=pltpu.CompilerParams(needs_layout_passes=False,
                                         use_tc_tiling_on_sc=False),
    scratch_types=[
        pltpu.VMEM((NB, B), jnp.int32),
        pltpu.VMEM((NB, B), jnp.int32),
        pltpu.VMEM((G, K, B, D), jnp.float32),
        pltpu.VMEM((RPT, D), jnp.float32),
        pltpu.VMEM_SHARED((NPAD, D), jnp.float32),
        pltpu.VMEM_SHARED((NPAD, D), jnp.float32),
    ] + [pltpu.SemaphoreType.DMA] * 8,
)(_sc_scatter_body)


def _tc_prep1_body(hT_ref, x_ref, W1_ref, dinv_ref, g1_ref):
    deg = jnp.sum(hT_ref[...], axis=1, keepdims=True) + 1.0
    dinv = lax.rsqrt(deg)
    dinv_ref[...] = dinv
    h = jnp.dot(x_ref[...], W1_ref[...], preferred_element_type=jnp.float32)
    g1_ref[...] = dinv * h


def _tc_mid_body(s1_ref, g1_ref, dinv_ref, b1_ref, W2_ref, g2_ref):
    dinv = dinv_ref[...]
    ssum = s1_ref[0] + s1_ref[1] + g1_ref[...]
    out1 = jnp.maximum(dinv * ssum + b1_ref[...], 0.0)
    g2_ref[...] = dinv * jnp.dot(out1, W2_ref[...],
                                 preferred_element_type=jnp.float32)


def _tc_final_body(s2_ref, g2_ref, dinv_ref, b2_ref, o_ref):
    o = dinv_ref[...] * (s2_ref[0] + s2_ref[1] + g2_ref[...]) + b2_ref[...]
    mask = lax.broadcasted_iota(jnp.int32, (NPAD, D), 1) < C
    om = jnp.where(mask, o, -jnp.inf)
    m = jnp.max(om, axis=1, keepdims=True)
    z = om - m
    e = jnp.where(mask, jnp.exp(z), 0.0)
    o_ref[...] = z - jnp.log(jnp.sum(e, axis=1, keepdims=True))


_tc_prep1 = pl.pallas_call(
    _tc_prep1_body,
    out_shape=(jax.ShapeDtypeStruct((NPAD, 1), jnp.float32),
               jax.ShapeDtypeStruct((NPAD, D), jnp.float32)),
)

_tc_mid = pl.pallas_call(
    _tc_mid_body,
    out_shape=jax.ShapeDtypeStruct((NPAD, D), jnp.float32),
)

_tc_final = pl.pallas_call(
    _tc_final_body,
    out_shape=jax.ShapeDtypeStruct((NPAD, D), jnp.float32),
)


def kernel(x, edge_index, W1, b1, W2, b2):
    src = edge_index[0].reshape(NW, EPT_RAW)
    dst = edge_index[1].reshape(NW, EPT_RAW)
    pad = EPT - EPT_RAW
    src_p = jnp.pad(src, ((0, 0), (0, pad)))
    dst_p = jnp.pad(dst, ((0, 0), (0, pad)), constant_values=SINK)
    src_t = src_p.reshape(NW, NB, B)
    dst_t = dst_p.reshape(NW, NB, B)

    xp = jnp.pad(x, ((0, NPAD - N), (0, 0)))
    W2p = jnp.pad(W2, ((0, 0), (0, D - C)))
    b1r = b1.reshape(1, D)
    b2r = jnp.pad(b2, (0, D - C)).reshape(1, D)

    hists = _sc_deg(dst_p)
    dinv, g1 = _tc_prep1(hists.T, xp, W1)
    s1 = _sc_scatter(g1, src_t, dst_t)
    g2 = _tc_mid(s1, g1, dinv, b1r, W2p)
    s2 = _sc_scatter(g2, src_t, dst_t)
    o = _tc_final(s2, g2, dinv, b2r)
    return o[:N, :C]
